# Initial kernel scaffold; baseline (speedup 1.0000x reference)
import jax
import jax.numpy as jnp
from jax.experimental import pallas as pl


def kernel(probability_distribution):
    raise NotImplementedError("write your pallas kernel here")


# speedup vs baseline: 1.6367x; 1.6367x over previous
import jax
import jax.numpy as jnp
from jax.experimental import pallas as pl
from jax.experimental.pallas import tpu as pltpu

BATCH = 1024
IN = 100000
OUT = 1024

PER_STEP = 4
CHUNK = PER_STEP * OUT
NK = (IN + CHUNK - 1) // CHUNK
BBLK = 256
NB = BATCH // BBLK


def _body(x_ref, o_ref):
    k = pl.program_id(1)
    x = x_ref[...]
    col = k * CHUNK + jax.lax.broadcasted_iota(jnp.int32, (BBLK, CHUNK), 1)
    x = jnp.where(col < IN, x, 0.0)
    acc = x[:, 0:OUT]
    for p in range(1, PER_STEP):
        acc = acc + x[:, p * OUT:(p + 1) * OUT]

    @pl.when(k == 0)
    def _init():
        o_ref[...] = acc

    @pl.when(k > 0)
    def _accum():
        o_ref[...] += acc


@jax.jit
def kernel(probability_distribution):
    return pl.pallas_call(
        _body,
        grid=(NB, NK),
        in_specs=[pl.BlockSpec((BBLK, CHUNK), lambda i, k: (i, k))],
        out_specs=pl.BlockSpec((BBLK, OUT), lambda i, k: (i, 0)),
        out_shape=jax.ShapeDtypeStruct((BATCH, OUT), jnp.float32),
        compiler_params=pltpu.CompilerParams(
            dimension_semantics=("parallel", "arbitrary"),
        ),
    )(probability_distribution)


# speedup vs baseline: 1.7194x; 1.0505x over previous
import jax
import jax.numpy as jnp
from jax.experimental import pallas as pl
from jax.experimental.pallas import tpu as pltpu

BATCH = 1024
IN = 100000
OUT = 1024

PER_STEP = 8
CHUNK = PER_STEP * OUT
NK = (IN + CHUNK - 1) // CHUNK
BBLK = 512
NB = BATCH // BBLK


def _reduce(x):
    acc = x[:, 0:OUT]
    for p in range(1, PER_STEP):
        acc = acc + x[:, p * OUT:(p + 1) * OUT]
    return acc


def _body(x_ref, o_ref):
    k = pl.program_id(1)

    @pl.when(k == 0)
    def _init():
        o_ref[...] = _reduce(x_ref[...])

    @pl.when(jnp.logical_and(k > 0, k < NK - 1))
    def _accum():
        o_ref[...] += _reduce(x_ref[...])

    @pl.when(k == NK - 1)
    def _tail():
        x = x_ref[...]
        col = k * CHUNK + jax.lax.broadcasted_iota(jnp.int32, (BBLK, CHUNK), 1)
        o_ref[...] += _reduce(jnp.where(col < IN, x, 0.0))


@jax.jit
def kernel(probability_distribution):
    return pl.pallas_call(
        _body,
        grid=(NB, NK),
        in_specs=[pl.BlockSpec((BBLK, CHUNK), lambda i, k: (i, k))],
        out_specs=pl.BlockSpec((BBLK, OUT), lambda i, k: (i, 0)),
        out_shape=jax.ShapeDtypeStruct((BATCH, OUT), jnp.float32),
        compiler_params=pltpu.CompilerParams(
            dimension_semantics=("parallel", "arbitrary"),
        ),
    )(probability_distribution)
